# Initial kernel scaffold; baseline (speedup 1.0000x reference)
import jax
import jax.numpy as jnp
from jax.experimental import pallas as pl


def kernel(point_cloud, Wf_pos, Wd_pos, g_pos, b_pos, Wf1, Wd1, g1, b1, g_bn1, b_bn1, Wf2, Wd2, g2, b2):
    raise NotImplementedError("write your pallas kernel here")


# speedup vs baseline: 25.7485x; 25.7485x over previous
import jax
import jax.numpy as jnp
from jax.experimental import pallas as pl

_EPS = 1e-6
_BN_EPS = 1e-5


def _vn_bn(p, g_col, b_col):
    norm = jnp.sqrt(p[0] * p[0] + p[1] * p[1] + p[2] * p[2]) + _EPS
    norm_bn = norm * (1.0 / jnp.sqrt(1.0 + _BN_EPS)) * g_col + b_col
    scale = norm_bn / norm
    return [p[0] * scale, p[1] * scale, p[2] * scale]


def _vn_lrelu(p, d):
    dot = p[0] * d[0] + p[1] * d[1] + p[2] * d[2]
    dnsq = d[0] * d[0] + d[1] * d[1] + d[2] * d[2]
    coef = jnp.where(dot >= 0.0, 0.0, dot / (dnsq + _EPS))
    return [p[0] - coef * d[0], p[1] - coef * d[1], p[2] - coef * d[2]]


def _kernel(x_ref, xt_ref, wfp_ref, wdp_ref, gp_ref, bp_ref,
            wf1_ref, wd1_ref, g1_ref, b1_ref, gbn1_ref, bbn1_ref,
            wf2_ref, wd2_ref, g2_ref, b2_ref, out_ref):
    f32 = jnp.float32
    X = x_ref[0]
    Xt = xt_ref[0]
    N = X.shape[1]

    Xb = X.astype(jnp.bfloat16)
    G = jax.lax.dot_general(Xb, Xb, (((0,), (0,)), ((), ())),
                            preferred_element_type=f32)
    xx_row = jnp.sum(X * X, axis=0, keepdims=True)
    xx_col = jnp.sum(Xt * Xt, axis=1, keepdims=True)
    t = (xx_col - 2.0 * G) + xx_row

    iota_j = jax.lax.broadcasted_iota(jnp.int32, (N, N), 0)
    m1 = jnp.max(t, axis=0, keepdims=True)
    j1 = jnp.min(jnp.where(t == m1, iota_j, N), axis=0, keepdims=True)
    hit1 = iota_j == j1
    t2 = jnp.where(hit1, -jnp.inf, t)
    m2 = jnp.max(t2, axis=0, keepdims=True)
    j2 = jnp.min(jnp.where(t2 == m2, iota_j, N), axis=0, keepdims=True)
    oh1 = hit1.astype(f32)
    oh2 = (iota_j == j2).astype(f32)

    F1 = jnp.dot(X, oh1, preferred_element_type=f32,
                 precision=jax.lax.Precision.HIGHEST)
    F2 = jnp.dot(X, oh2, preferred_element_type=f32,
                 precision=jax.lax.Precision.HIGHEST)

    xs = [X[0:1, :], X[1:2, :], X[2:3, :]]
    Wfp = wfp_ref[...]
    Wdp = wdp_ref[...]
    gp = gp_ref[...]
    bp = bp_ref[...]

    def layer1(F):
        fs = [F[0:1, :], F[1:2, :], F[2:3, :]]
        c0 = [fs[v] - xs[v] for v in range(3)]
        c1 = xs
        c2 = [fs[1] * xs[2] - fs[2] * xs[1],
              fs[2] * xs[0] - fs[0] * xs[2],
              fs[0] * xs[1] - fs[1] * xs[0]]
        rb = lambda a: a.astype(jnp.bfloat16).astype(jnp.float32)
        c0 = [rb(c) for c in c0]
        c1r = [rb(c) for c in c1]
        c2 = [rb(c) for c in c2]
        p = [rb(Wfp[:, 0:1]) * c0[v] + rb(Wfp[:, 1:2]) * c1r[v]
             + rb(Wfp[:, 2:3]) * c2[v] for v in range(3)]
        d = [rb(Wdp[:, 0:1]) * c0[v] + rb(Wdp[:, 1:2]) * c1r[v]
             + rb(Wdp[:, 2:3]) * c2[v] for v in range(3)]
        p = _vn_bn(p, gp, bp)
        return _vn_lrelu(p, d)

    y1 = layer1(F1)
    y2 = layer1(F2)
    y = [(y1[v] + y2[v]) * 0.5 for v in range(3)]

    Wf1 = wf1_ref[...]
    Wd1 = wd1_ref[...]
    p = [jnp.dot(Wf1, y[v], preferred_element_type=f32) for v in range(3)]
    d = [jnp.dot(Wd1, y[v], preferred_element_type=f32) for v in range(3)]
    p = _vn_bn(p, g1_ref[...], b1_ref[...])
    z = _vn_lrelu(p, d)
    z = _vn_bn(z, gbn1_ref[...], bbn1_ref[...])

    Wf2 = wf2_ref[...]
    Wd2 = wd2_ref[...]
    p = [jnp.dot(Wf2, z[v], preferred_element_type=f32) for v in range(3)]
    d = [jnp.dot(Wd2, z[v], preferred_element_type=f32) for v in range(3)]
    p = _vn_bn(p, g2_ref[...], b2_ref[...])
    w = _vn_lrelu(p, d)
    inv_n = 1.0 / N
    cols = [jnp.sum(w[v], axis=1, keepdims=True) * inv_n for v in range(3)]
    out_ref[0] = jnp.concatenate(cols, axis=1)


def kernel(point_cloud, Wf_pos, Wd_pos, g_pos, b_pos, Wf1, Wd1, g1, b1,
           g_bn1, b_bn1, Wf2, Wd2, g2, b2):
    B, N, _ = point_cloud.shape
    pc = jnp.swapaxes(point_cloud, 2, 1)
    C = Wf_pos.shape[0]
    CO = Wf2.shape[0]

    def col(v):
        return v.reshape(-1, 1)

    full = lambda a: pl.BlockSpec(a.shape, lambda b: (0,) * a.ndim)
    args = (pc, point_cloud, Wf_pos, Wd_pos, col(g_pos), col(b_pos),
            Wf1, Wd1, col(g1), col(b1), col(g_bn1), col(b_bn1), Wf2, Wd2,
            col(g2), col(b2))
    out = pl.pallas_call(
        _kernel,
        grid=(B,),
        in_specs=[pl.BlockSpec((1, 3, N), lambda b: (b, 0, 0)),
                  pl.BlockSpec((1, N, 3), lambda b: (b, 0, 0))]
        + [full(a) for a in args[2:]],
        out_specs=pl.BlockSpec((1, CO, 3), lambda b: (b, 0, 0)),
        out_shape=jax.ShapeDtypeStruct((B, CO, 3), jnp.float32),
    )(*args)
    return out[:, :3]


# speedup vs baseline: 35.8814x; 1.3935x over previous
import jax
import jax.numpy as jnp
from jax.experimental import pallas as pl

_EPS = 1e-6
_BN_EPS = 1e-5


def _vn_bn(p, g_col, b_col):
    norm = jnp.sqrt(p[0] * p[0] + p[1] * p[1] + p[2] * p[2]) + _EPS
    norm_bn = norm * (1.0 / jnp.sqrt(1.0 + _BN_EPS)) * g_col + b_col
    scale = norm_bn / norm
    return [p[0] * scale, p[1] * scale, p[2] * scale]


def _vn_lrelu(p, d):
    dot = p[0] * d[0] + p[1] * d[1] + p[2] * d[2]
    dnsq = d[0] * d[0] + d[1] * d[1] + d[2] * d[2]
    coef = jnp.where(dot >= 0.0, 0.0, dot / (dnsq + _EPS))
    return [p[0] - coef * d[0], p[1] - coef * d[1], p[2] - coef * d[2]]


def _kernel(x_ref, xt_ref, wfp_ref, wdp_ref, gp_ref, bp_ref,
            wf1_ref, wd1_ref, g1_ref, b1_ref, gbn1_ref, bbn1_ref,
            wf2_ref, wd2_ref, g2_ref, b2_ref, out_ref):
    f32 = jnp.float32
    X = x_ref[0]
    Xt = xt_ref[0]
    N = X.shape[1]

    Xb = X.astype(jnp.bfloat16)
    G = jax.lax.dot_general(Xb, Xb, (((0,), (0,)), ((), ())),
                            preferred_element_type=f32)
    xx_row = jnp.sum(X * X, axis=0, keepdims=True)
    xx_col = jnp.sum(Xt * Xt, axis=1, keepdims=True)
    t = (xx_col - 2.0 * G) + xx_row

    bf16 = jnp.bfloat16
    iota_j = jax.lax.broadcasted_iota(jnp.int32, (N, N), 0)
    m1 = jnp.max(t, axis=0, keepdims=True)
    j1 = jnp.min(jnp.where(t == m1, iota_j, N), axis=0, keepdims=True)
    hit1 = iota_j == j1
    t2 = jnp.where(hit1, -jnp.inf, t)
    m2 = jnp.max(t2, axis=0, keepdims=True)
    j2 = jnp.min(jnp.where(t2 == m2, iota_j, N), axis=0, keepdims=True)
    oh1 = hit1.astype(f32).astype(bf16)
    oh2 = (iota_j == j2).astype(f32).astype(bf16)

    Xp1 = X.astype(bf16)
    r1 = X - Xp1.astype(f32)
    Xp2 = r1.astype(bf16)
    Xp3 = (r1 - Xp2.astype(f32)).astype(bf16)

    def gather(oh):
        acc = jnp.dot(Xp1, oh, preferred_element_type=f32)
        acc = acc + jnp.dot(Xp2, oh, preferred_element_type=f32)
        return acc + jnp.dot(Xp3, oh, preferred_element_type=f32)

    F1 = gather(oh1)
    F2 = gather(oh2)

    xs = [X[0:1, :], X[1:2, :], X[2:3, :]]
    Wfp = wfp_ref[...]
    Wdp = wdp_ref[...]
    gp = gp_ref[...]
    bp = bp_ref[...]

    def layer1(F):
        fs = [F[0:1, :], F[1:2, :], F[2:3, :]]
        c0 = [fs[v] - xs[v] for v in range(3)]
        c1 = xs
        c2 = [fs[1] * xs[2] - fs[2] * xs[1],
              fs[2] * xs[0] - fs[0] * xs[2],
              fs[0] * xs[1] - fs[1] * xs[0]]
        rb = lambda a: a.astype(jnp.bfloat16).astype(jnp.float32)
        c0 = [rb(c) for c in c0]
        c1r = [rb(c) for c in c1]
        c2 = [rb(c) for c in c2]
        p = [rb(Wfp[:, 0:1]) * c0[v] + rb(Wfp[:, 1:2]) * c1r[v]
             + rb(Wfp[:, 2:3]) * c2[v] for v in range(3)]
        d = [rb(Wdp[:, 0:1]) * c0[v] + rb(Wdp[:, 1:2]) * c1r[v]
             + rb(Wdp[:, 2:3]) * c2[v] for v in range(3)]
        p = _vn_bn(p, gp, bp)
        return _vn_lrelu(p, d)

    y1 = layer1(F1)
    y2 = layer1(F2)
    y = [(y1[v] + y2[v]) * 0.5 for v in range(3)]

    Wf1 = wf1_ref[...]
    Wd1 = wd1_ref[...]
    p = [jnp.dot(Wf1, y[v], preferred_element_type=f32) for v in range(3)]
    d = [jnp.dot(Wd1, y[v], preferred_element_type=f32) for v in range(3)]
    p = _vn_bn(p, g1_ref[...], b1_ref[...])
    z = _vn_lrelu(p, d)
    z = _vn_bn(z, gbn1_ref[...], bbn1_ref[...])

    Wf2 = wf2_ref[...]
    Wd2 = wd2_ref[...]
    p = [jnp.dot(Wf2, z[v], preferred_element_type=f32) for v in range(3)]
    d = [jnp.dot(Wd2, z[v], preferred_element_type=f32) for v in range(3)]
    p = _vn_bn(p, g2_ref[...], b2_ref[...])
    w = _vn_lrelu(p, d)
    inv_n = 1.0 / N
    cols = [jnp.sum(w[v], axis=1, keepdims=True) * inv_n for v in range(3)]
    out_ref[0] = jnp.concatenate(cols, axis=1)


def kernel(point_cloud, Wf_pos, Wd_pos, g_pos, b_pos, Wf1, Wd1, g1, b1,
           g_bn1, b_bn1, Wf2, Wd2, g2, b2):
    B, N, _ = point_cloud.shape
    pc = jnp.swapaxes(point_cloud, 2, 1)
    C = Wf_pos.shape[0]
    CO = Wf2.shape[0]

    def col(v):
        return v.reshape(-1, 1)

    full = lambda a: pl.BlockSpec(a.shape, lambda b: (0,) * a.ndim)
    args = (pc, point_cloud, Wf_pos, Wd_pos, col(g_pos), col(b_pos),
            Wf1, Wd1, col(g1), col(b1), col(g_bn1), col(b_bn1), Wf2, Wd2,
            col(g2), col(b2))
    out = pl.pallas_call(
        _kernel,
        grid=(B,),
        in_specs=[pl.BlockSpec((1, 3, N), lambda b: (b, 0, 0)),
                  pl.BlockSpec((1, N, 3), lambda b: (b, 0, 0))]
        + [full(a) for a in args[2:]],
        out_specs=pl.BlockSpec((1, CO, 3), lambda b: (b, 0, 0)),
        out_shape=jax.ShapeDtypeStruct((B, CO, 3), jnp.float32),
    )(*args)
    return out[:, :3]


# speedup vs baseline: 44.0447x; 1.2275x over previous
import jax
import jax.numpy as jnp
from jax.experimental import pallas as pl

_EPS = 1e-6
_BN_EPS = 1e-5


def _vn_bn(p, g_col, b_col):
    norm = jnp.sqrt(p[0] * p[0] + p[1] * p[1] + p[2] * p[2]) + _EPS
    norm_bn = norm * (1.0 / jnp.sqrt(1.0 + _BN_EPS)) * g_col + b_col
    scale = norm_bn / norm
    return [p[0] * scale, p[1] * scale, p[2] * scale]


def _vn_lrelu(p, d):
    dot = p[0] * d[0] + p[1] * d[1] + p[2] * d[2]
    dnsq = d[0] * d[0] + d[1] * d[1] + d[2] * d[2]
    coef = jnp.where(dot >= 0.0, 0.0, dot / (dnsq + _EPS))
    return [p[0] - coef * d[0], p[1] - coef * d[1], p[2] - coef * d[2]]


def _kernel(x_ref, xt_ref, wfp_ref, wdp_ref, gp_ref, bp_ref,
            wf1_ref, wd1_ref, g1_ref, b1_ref, gbn1_ref, bbn1_ref,
            wf2_ref, wd2_ref, g2_ref, b2_ref, out_ref):
    f32 = jnp.float32
    X = x_ref[0]
    Xt = xt_ref[0]
    N = X.shape[1]

    Xb = X.astype(jnp.bfloat16)
    Gm2 = jax.lax.dot_general((-2.0 * X).astype(jnp.bfloat16), Xb,
                              (((0,), (0,)), ((), ())),
                              preferred_element_type=f32)
    xx_row = jnp.sum(X * X, axis=0, keepdims=True)
    xx_col = jnp.sum(Xt * Xt, axis=1, keepdims=True)
    t = (xx_col + Gm2) + xx_row

    bf16 = jnp.bfloat16
    iota_j = jax.lax.broadcasted_iota(jnp.int32, (N, N), 0)
    m1 = jnp.max(t, axis=0, keepdims=True)
    j1 = jnp.min(jnp.where(t == m1, iota_j, N), axis=0, keepdims=True)
    hit1 = iota_j == j1
    t2 = jnp.where(hit1, -jnp.inf, t)
    m2 = jnp.max(t2, axis=0, keepdims=True)
    j2 = jnp.min(jnp.where(t2 == m2, iota_j, N), axis=0, keepdims=True)
    oh1 = hit1.astype(f32).astype(bf16)
    oh2 = (iota_j == j2).astype(f32).astype(bf16)

    Xp1 = X.astype(bf16)
    r1 = X - Xp1.astype(f32)
    Xp2 = r1.astype(bf16)
    Xp3 = (r1 - Xp2.astype(f32)).astype(bf16)
    Xcat = jnp.concatenate([Xp1, Xp2, Xp3], axis=0)

    def gather(oh):
        g = jnp.dot(Xcat, oh, preferred_element_type=f32)
        return (g[0:3, :] + g[3:6, :]) + g[6:9, :]

    F1 = gather(oh1)
    F2 = gather(oh2)

    xs = [X[0:1, :], X[1:2, :], X[2:3, :]]
    Wfp = wfp_ref[...]
    Wdp = wdp_ref[...]
    gp = gp_ref[...]
    bp = bp_ref[...]

    def layer1(F):
        fs = [F[0:1, :], F[1:2, :], F[2:3, :]]
        c0 = [fs[v] - xs[v] for v in range(3)]
        c1 = xs
        c2 = [fs[1] * xs[2] - fs[2] * xs[1],
              fs[2] * xs[0] - fs[0] * xs[2],
              fs[0] * xs[1] - fs[1] * xs[0]]
        rb = lambda a: a.astype(jnp.bfloat16).astype(jnp.float32)
        c0 = [rb(c) for c in c0]
        c1r = [rb(c) for c in c1]
        c2 = [rb(c) for c in c2]
        p = [rb(Wfp[:, 0:1]) * c0[v] + rb(Wfp[:, 1:2]) * c1r[v]
             + rb(Wfp[:, 2:3]) * c2[v] for v in range(3)]
        d = [rb(Wdp[:, 0:1]) * c0[v] + rb(Wdp[:, 1:2]) * c1r[v]
             + rb(Wdp[:, 2:3]) * c2[v] for v in range(3)]
        p = _vn_bn(p, gp, bp)
        return _vn_lrelu(p, d)

    y1 = layer1(F1)
    y2 = layer1(F2)
    y = [(y1[v] + y2[v]) * 0.5 for v in range(3)]

    Wf1 = wf1_ref[...]
    Wd1 = wd1_ref[...]
    p = [jnp.dot(Wf1, y[v], preferred_element_type=f32) for v in range(3)]
    d = [jnp.dot(Wd1, y[v], preferred_element_type=f32) for v in range(3)]
    p = _vn_bn(p, g1_ref[...], b1_ref[...])
    z = _vn_lrelu(p, d)
    z = _vn_bn(z, gbn1_ref[...], bbn1_ref[...])

    Wf2 = wf2_ref[...]
    Wd2 = wd2_ref[...]
    p = [jnp.dot(Wf2, z[v], preferred_element_type=f32) for v in range(3)]
    d = [jnp.dot(Wd2, z[v], preferred_element_type=f32) for v in range(3)]
    p = _vn_bn(p, g2_ref[...], b2_ref[...])
    w = _vn_lrelu(p, d)
    inv_n = 1.0 / N
    cols = [jnp.sum(w[v], axis=1, keepdims=True) * inv_n for v in range(3)]
    out_ref[0] = jnp.concatenate(cols, axis=1)


def kernel(point_cloud, Wf_pos, Wd_pos, g_pos, b_pos, Wf1, Wd1, g1, b1,
           g_bn1, b_bn1, Wf2, Wd2, g2, b2):
    B, N, _ = point_cloud.shape
    pc = jnp.swapaxes(point_cloud, 2, 1)
    C = Wf_pos.shape[0]
    CO = Wf2.shape[0]

    def col(v):
        return v.reshape(-1, 1)

    full = lambda a: pl.BlockSpec(a.shape, lambda b: (0,) * a.ndim)
    args = (pc, point_cloud, Wf_pos, Wd_pos, col(g_pos), col(b_pos),
            Wf1, Wd1, col(g1), col(b1), col(g_bn1), col(b_bn1), Wf2, Wd2,
            col(g2), col(b2))
    out = pl.pallas_call(
        _kernel,
        grid=(B,),
        in_specs=[pl.BlockSpec((1, 3, N), lambda b: (b, 0, 0)),
                  pl.BlockSpec((1, N, 3), lambda b: (b, 0, 0))]
        + [full(a) for a in args[2:]],
        out_specs=pl.BlockSpec((1, CO, 3), lambda b: (b, 0, 0)),
        out_shape=jax.ShapeDtypeStruct((B, CO, 3), jnp.float32),
    )(*args)
    return out[:, :3]


# speedup vs baseline: 46.3678x; 1.0527x over previous
import jax
import jax.numpy as jnp
from jax.experimental import pallas as pl

_EPS = 1e-6
_BN_EPS = 1e-5


def _vn_bn(p, g_col, b_col):
    norm = jnp.sqrt(p[0] * p[0] + p[1] * p[1] + p[2] * p[2]) + _EPS
    norm_bn = norm * (1.0 / jnp.sqrt(1.0 + _BN_EPS)) * g_col + b_col
    scale = norm_bn / norm
    return [p[0] * scale, p[1] * scale, p[2] * scale]


def _vn_lrelu(p, d):
    dot = p[0] * d[0] + p[1] * d[1] + p[2] * d[2]
    dnsq = d[0] * d[0] + d[1] * d[1] + d[2] * d[2]
    coef = jnp.where(dot >= 0.0, 0.0, dot / (dnsq + _EPS))
    return [p[0] - coef * d[0], p[1] - coef * d[1], p[2] - coef * d[2]]


def _kernel(x_ref, xt_ref, wfp_ref, wdp_ref, gp_ref, bp_ref,
            wf1_ref, wd1_ref, g1_ref, b1_ref, gbn1_ref, bbn1_ref,
            wf2_ref, wd2_ref, g2_ref, b2_ref, out_ref):
    f32 = jnp.float32
    X = x_ref[0]
    Xt = xt_ref[0]
    N = X.shape[1]

    Xb = X.astype(jnp.bfloat16)
    Gm2 = jax.lax.dot_general((-2.0 * X).astype(jnp.bfloat16), Xb,
                              (((0,), (0,)), ((), ())),
                              preferred_element_type=f32)
    xx_row = jnp.sum(X * X, axis=0, keepdims=True)
    xx_col = jnp.sum(Xt * Xt, axis=1, keepdims=True)
    t = (xx_col + Gm2) + xx_row

    bf16 = jnp.bfloat16
    iota_f = jax.lax.broadcasted_iota(jnp.int32, (N, N), 0).astype(f32)
    m1 = jnp.max(t, axis=0, keepdims=True)
    j1 = jnp.min(jnp.where(t == m1, iota_f, f32(N)), axis=0, keepdims=True)
    hit1 = iota_f == j1
    t2 = jnp.where(hit1, -jnp.inf, t)
    m2 = jnp.max(t2, axis=0, keepdims=True)
    j2 = jnp.min(jnp.where(t2 == m2, iota_f, f32(N)), axis=0, keepdims=True)
    oh1 = hit1.astype(f32).astype(bf16)
    oh2 = (iota_f == j2).astype(f32).astype(bf16)

    Xp1 = X.astype(bf16)
    r1 = X - Xp1.astype(f32)
    Xp2 = r1.astype(bf16)
    Xp3 = (r1 - Xp2.astype(f32)).astype(bf16)
    Xcat = jnp.concatenate([Xp1, Xp2, Xp3], axis=0)

    def gather(oh):
        g = jnp.dot(Xcat, oh, preferred_element_type=f32)
        return (g[0:3, :] + g[3:6, :]) + g[6:9, :]

    F1 = gather(oh1)
    F2 = gather(oh2)

    xs = [X[0:1, :], X[1:2, :], X[2:3, :]]
    Wfp = wfp_ref[...]
    Wdp = wdp_ref[...]
    gp = gp_ref[...]
    bp = bp_ref[...]

    def layer1(F):
        fs = [F[0:1, :], F[1:2, :], F[2:3, :]]
        c0 = [fs[v] - xs[v] for v in range(3)]
        c1 = xs
        c2 = [fs[1] * xs[2] - fs[2] * xs[1],
              fs[2] * xs[0] - fs[0] * xs[2],
              fs[0] * xs[1] - fs[1] * xs[0]]
        rb = lambda a: a.astype(jnp.bfloat16).astype(jnp.float32)
        c0 = [rb(c) for c in c0]
        c1r = [rb(c) for c in c1]
        c2 = [rb(c) for c in c2]
        p = [rb(Wfp[:, 0:1]) * c0[v] + rb(Wfp[:, 1:2]) * c1r[v]
             + rb(Wfp[:, 2:3]) * c2[v] for v in range(3)]
        d = [rb(Wdp[:, 0:1]) * c0[v] + rb(Wdp[:, 1:2]) * c1r[v]
             + rb(Wdp[:, 2:3]) * c2[v] for v in range(3)]
        p = _vn_bn(p, gp, bp)
        return _vn_lrelu(p, d)

    y1 = layer1(F1)
    y2 = layer1(F2)
    y = [(y1[v] + y2[v]) * 0.5 for v in range(3)]

    Wf1 = wf1_ref[...]
    Wd1 = wd1_ref[...]
    p = [jnp.dot(Wf1, y[v], preferred_element_type=f32) for v in range(3)]
    d = [jnp.dot(Wd1, y[v], preferred_element_type=f32) for v in range(3)]
    p = _vn_bn(p, g1_ref[...], b1_ref[...])
    z = _vn_lrelu(p, d)
    z = _vn_bn(z, gbn1_ref[...], bbn1_ref[...])

    Wf2 = wf2_ref[...]
    Wd2 = wd2_ref[...]
    p = [jnp.dot(Wf2, z[v], preferred_element_type=f32) for v in range(3)]
    d = [jnp.dot(Wd2, z[v], preferred_element_type=f32) for v in range(3)]
    p = _vn_bn(p, g2_ref[...], b2_ref[...])
    w = _vn_lrelu(p, d)
    inv_n = 1.0 / N
    cols = [jnp.sum(w[v], axis=1, keepdims=True) * inv_n for v in range(3)]
    out_ref[0] = jnp.concatenate(cols, axis=1)


def kernel(point_cloud, Wf_pos, Wd_pos, g_pos, b_pos, Wf1, Wd1, g1, b1,
           g_bn1, b_bn1, Wf2, Wd2, g2, b2):
    B, N, _ = point_cloud.shape
    pc = jnp.swapaxes(point_cloud, 2, 1)
    C = Wf_pos.shape[0]
    CO = Wf2.shape[0]

    def col(v):
        return v.reshape(-1, 1)

    full = lambda a: pl.BlockSpec(a.shape, lambda b: (0,) * a.ndim)
    args = (pc, point_cloud, Wf_pos, Wd_pos, col(g_pos), col(b_pos),
            Wf1, Wd1, col(g1), col(b1), col(g_bn1), col(b_bn1), Wf2, Wd2,
            col(g2), col(b2))
    out = pl.pallas_call(
        _kernel,
        grid=(B,),
        in_specs=[pl.BlockSpec((1, 3, N), lambda b: (b, 0, 0)),
                  pl.BlockSpec((1, N, 3), lambda b: (b, 0, 0))]
        + [full(a) for a in args[2:]],
        out_specs=pl.BlockSpec((1, CO, 3), lambda b: (b, 0, 0)),
        out_shape=jax.ShapeDtypeStruct((B, CO, 3), jnp.float32),
    )(*args)
    return out[:, :3]


# speedup vs baseline: 47.2626x; 1.0193x over previous
import jax
import jax.numpy as jnp
from jax.experimental import pallas as pl

_EPS = 1e-6
_BN_EPS = 1e-5


def _vn_bn(p, g_col, b_col):
    norm = jnp.sqrt(p[0] * p[0] + p[1] * p[1] + p[2] * p[2]) + _EPS
    norm_bn = norm * (1.0 / jnp.sqrt(1.0 + _BN_EPS)) * g_col + b_col
    scale = norm_bn / norm
    return [p[0] * scale, p[1] * scale, p[2] * scale]


def _vn_lrelu(p, d):
    dot = p[0] * d[0] + p[1] * d[1] + p[2] * d[2]
    dnsq = d[0] * d[0] + d[1] * d[1] + d[2] * d[2]
    coef = jnp.where(dot >= 0.0, 0.0, dot / (dnsq + _EPS))
    return [p[0] - coef * d[0], p[1] - coef * d[1], p[2] - coef * d[2]]


def _kernel(x_ref, xt_ref, wfp_ref, wdp_ref, gp_ref, bp_ref,
            wf1_ref, wd1_ref, g1_ref, b1_ref, gbn1_ref, bbn1_ref,
            wf2_ref, wd2_ref, g2_ref, b2_ref, out_ref):
    for blk in range(x_ref.shape[0]):
        _one_batch(x_ref[blk], xt_ref[blk], wfp_ref, wdp_ref, gp_ref,
                   bp_ref, wf1_ref, wd1_ref, g1_ref, b1_ref, gbn1_ref,
                   bbn1_ref, wf2_ref, wd2_ref, g2_ref, b2_ref, out_ref, blk)


def _one_batch(X, Xt, wfp_ref, wdp_ref, gp_ref, bp_ref,
               wf1_ref, wd1_ref, g1_ref, b1_ref, gbn1_ref, bbn1_ref,
               wf2_ref, wd2_ref, g2_ref, b2_ref, out_ref, blk):
    f32 = jnp.float32
    N = X.shape[1]

    Xb = X.astype(jnp.bfloat16)
    Gm2 = jax.lax.dot_general((-2.0 * X).astype(jnp.bfloat16), Xb,
                              (((0,), (0,)), ((), ())),
                              preferred_element_type=f32)
    xx_row = jnp.sum(X * X, axis=0, keepdims=True)
    xx_col = jnp.sum(Xt * Xt, axis=1, keepdims=True)
    t = (xx_col + Gm2) + xx_row

    bf16 = jnp.bfloat16
    iota_f = jax.lax.broadcasted_iota(jnp.int32, (N, N), 0).astype(f32)
    m1 = jnp.max(t, axis=0, keepdims=True)
    j1 = jnp.min(jnp.where(t == m1, iota_f, f32(N)), axis=0, keepdims=True)
    hit1 = iota_f == j1
    t2 = jnp.where(hit1, -jnp.inf, t)
    m2 = jnp.max(t2, axis=0, keepdims=True)
    j2 = jnp.min(jnp.where(t2 == m2, iota_f, f32(N)), axis=0, keepdims=True)
    oh1 = hit1.astype(f32).astype(bf16)
    oh2 = (iota_f == j2).astype(f32).astype(bf16)

    Xp1 = X.astype(bf16)
    r1 = X - Xp1.astype(f32)
    Xp2 = r1.astype(bf16)
    Xp3 = (r1 - Xp2.astype(f32)).astype(bf16)
    Xcat = jnp.concatenate([Xp1, Xp2, Xp3], axis=0)

    def gather(oh):
        g = jnp.dot(Xcat, oh, preferred_element_type=f32)
        return (g[0:3, :] + g[3:6, :]) + g[6:9, :]

    F1 = gather(oh1)
    F2 = gather(oh2)

    xs = [X[0:1, :], X[1:2, :], X[2:3, :]]
    Wfp = wfp_ref[...]
    Wdp = wdp_ref[...]
    gp = gp_ref[...]
    bp = bp_ref[...]

    def layer1(F):
        fs = [F[0:1, :], F[1:2, :], F[2:3, :]]
        c0 = [fs[v] - xs[v] for v in range(3)]
        c1 = xs
        c2 = [fs[1] * xs[2] - fs[2] * xs[1],
              fs[2] * xs[0] - fs[0] * xs[2],
              fs[0] * xs[1] - fs[1] * xs[0]]
        rb = lambda a: a.astype(jnp.bfloat16).astype(jnp.float32)
        c0 = [rb(c) for c in c0]
        c1r = [rb(c) for c in c1]
        c2 = [rb(c) for c in c2]
        p = [rb(Wfp[:, 0:1]) * c0[v] + rb(Wfp[:, 1:2]) * c1r[v]
             + rb(Wfp[:, 2:3]) * c2[v] for v in range(3)]
        d = [rb(Wdp[:, 0:1]) * c0[v] + rb(Wdp[:, 1:2]) * c1r[v]
             + rb(Wdp[:, 2:3]) * c2[v] for v in range(3)]
        p = _vn_bn(p, gp, bp)
        return _vn_lrelu(p, d)

    y1 = layer1(F1)
    y2 = layer1(F2)
    y = [(y1[v] + y2[v]) * 0.5 for v in range(3)]

    Wf1 = wf1_ref[...]
    Wd1 = wd1_ref[...]
    p = [jnp.dot(Wf1, y[v], preferred_element_type=f32) for v in range(3)]
    d = [jnp.dot(Wd1, y[v], preferred_element_type=f32) for v in range(3)]
    p = _vn_bn(p, g1_ref[...], b1_ref[...])
    z = _vn_lrelu(p, d)
    z = _vn_bn(z, gbn1_ref[...], bbn1_ref[...])

    Wf2 = wf2_ref[...]
    Wd2 = wd2_ref[...]
    p = [jnp.dot(Wf2, z[v], preferred_element_type=f32) for v in range(3)]
    d = [jnp.dot(Wd2, z[v], preferred_element_type=f32) for v in range(3)]
    p = _vn_bn(p, g2_ref[...], b2_ref[...])
    w = _vn_lrelu(p, d)
    inv_n = 1.0 / N
    cols = [jnp.sum(w[v], axis=1, keepdims=True) * inv_n for v in range(3)]
    out_ref[blk] = jnp.concatenate(cols, axis=1)


def kernel(point_cloud, Wf_pos, Wd_pos, g_pos, b_pos, Wf1, Wd1, g1, b1,
           g_bn1, b_bn1, Wf2, Wd2, g2, b2):
    B, N, _ = point_cloud.shape
    pc = jnp.swapaxes(point_cloud, 2, 1)
    C = Wf_pos.shape[0]
    CO = Wf2.shape[0]

    def col(v):
        return v.reshape(-1, 1)

    full = lambda a: pl.BlockSpec(a.shape, lambda b: (0,) * a.ndim)
    args = (pc, point_cloud, Wf_pos, Wd_pos, col(g_pos), col(b_pos),
            Wf1, Wd1, col(g1), col(b1), col(g_bn1), col(b_bn1), Wf2, Wd2,
            col(g2), col(b2))
    out = pl.pallas_call(
        _kernel,
        grid=(B // 2,),
        in_specs=[pl.BlockSpec((2, 3, N), lambda b: (b, 0, 0)),
                  pl.BlockSpec((2, N, 3), lambda b: (b, 0, 0))]
        + [full(a) for a in args[2:]],
        out_specs=pl.BlockSpec((2, CO, 3), lambda b: (b, 0, 0)),
        out_shape=jax.ShapeDtypeStruct((B, CO, 3), jnp.float32),
    )(*args)
    return out[:, :3]
